# baseline (speedup 1.0000x reference)
import jax
import jax.numpy as jnp
from jax.experimental import pallas as pl
from jax.experimental.pallas import tpu as pltpu

_LANES = 128
_ROWS = 50000
_BLOCK_ROWS = 2000


def _sigmoid_block(k_ref, b_ref, o_ref):
    o_ref[...] = jax.nn.sigmoid(b_ref[0])


def kernel(B, k, edge_index, n):
    K, E = B.shape
    b3 = B.reshape(K, _ROWS, _LANES)
    k_arr = jnp.atleast_1d(k).astype(jnp.int32)
    grid = _ROWS // _BLOCK_ROWS
    out = pl.pallas_call(
        _sigmoid_block,
        grid_spec=pltpu.PrefetchScalarGridSpec(
            num_scalar_prefetch=1,
            grid=(grid,),
            in_specs=[
                pl.BlockSpec((1, _BLOCK_ROWS, _LANES),
                             lambda i, kref: (kref[0], i, 0)),
            ],
            out_specs=pl.BlockSpec((_BLOCK_ROWS, _LANES),
                                   lambda i, kref: (i, 0)),
        ),
        out_shape=jax.ShapeDtypeStruct((_ROWS, _LANES), jnp.float32),
    )(k_arr, b3)
    return out.reshape(E)


# speedup vs baseline: 1.0746x; 1.0746x over previous
import jax
import jax.numpy as jnp
from jax.experimental import pallas as pl
from jax.experimental.pallas import tpu as pltpu

_LANES = 128
_ROWS = 50000
_BLOCK_ROWS = 5000


def _sigmoid_block(k_ref, b_ref, o_ref):
    o_ref[...] = jax.nn.sigmoid(b_ref[0])


def kernel(B, k, edge_index, n):
    K, E = B.shape
    b3 = B.reshape(K, _ROWS, _LANES)
    k_arr = jnp.atleast_1d(k).astype(jnp.int32)
    grid = _ROWS // _BLOCK_ROWS
    out = pl.pallas_call(
        _sigmoid_block,
        grid_spec=pltpu.PrefetchScalarGridSpec(
            num_scalar_prefetch=1,
            grid=(grid,),
            in_specs=[
                pl.BlockSpec((1, _BLOCK_ROWS, _LANES),
                             lambda i, kref: (kref[0], i, 0)),
            ],
            out_specs=pl.BlockSpec((_BLOCK_ROWS, _LANES),
                                   lambda i, kref: (i, 0)),
        ),
        out_shape=jax.ShapeDtypeStruct((_ROWS, _LANES), jnp.float32),
    )(k_arr, b3)
    return out.reshape(E)


# speedup vs baseline: 5.6669x; 5.2733x over previous
import jax
import jax.numpy as jnp
from jax.experimental import pallas as pl
from jax.experimental.pallas import tpu as pltpu

_CHUNK = 640000
_NSTEPS = 10


def _body(k_ref, b_hbm, o_ref, scratch, sems):
    i = pl.program_id(0)
    k = k_ref[0]
    slot = jax.lax.rem(i, 2)
    nxt = jax.lax.rem(i + 1, 2)

    @pl.when(i == 0)
    def _first():
        pltpu.make_async_copy(
            b_hbm.at[k, pl.ds(0, _CHUNK)], scratch.at[0], sems.at[0]
        ).start()

    @pl.when(i + 1 < _NSTEPS)
    def _prefetch():
        pltpu.make_async_copy(
            b_hbm.at[k, pl.ds((i + 1) * _CHUNK, _CHUNK)],
            scratch.at[nxt], sems.at[nxt],
        ).start()

    pltpu.make_async_copy(
        b_hbm.at[k, pl.ds(i * _CHUNK, _CHUNK)], scratch.at[slot], sems.at[slot]
    ).wait()
    o_ref[...] = jax.nn.sigmoid(scratch[slot])


def kernel(B, k, edge_index, n):
    E = B.shape[1]
    k_arr = jnp.atleast_1d(k).astype(jnp.int32)
    out = pl.pallas_call(
        _body,
        grid_spec=pltpu.PrefetchScalarGridSpec(
            num_scalar_prefetch=1,
            grid=(_NSTEPS,),
            in_specs=[pl.BlockSpec(memory_space=pl.ANY)],
            out_specs=pl.BlockSpec((_CHUNK,), lambda i, kref: (i,)),
            scratch_shapes=[
                pltpu.VMEM((2, _CHUNK), jnp.float32),
                pltpu.SemaphoreType.DMA((2,)),
            ],
        ),
        out_shape=jax.ShapeDtypeStruct((E,), jnp.float32),
    )(k_arr, B)
    return out
